# Initial kernel scaffold; baseline (speedup 1.0000x reference)
import jax
import jax.numpy as jnp
from jax.experimental import pallas as pl


def kernel(x, edge_index, weight):
    raise NotImplementedError("write your pallas kernel here")


# speedup vs baseline: 5.4261x; 5.4261x over previous
import functools

import jax
import jax.numpy as jnp
from jax import lax
from jax.experimental import pallas as pl
from jax.experimental.pallas import tpu as pltpu
from jax.experimental.pallas import tpu_sc as plsc

N_CORES = 2
N_SUBCORES = 16
N_WORKERS = N_CORES * N_SUBCORES
CHUNK = 80


def _gemm_body(x_ref, w_ref, o_ref):
    o_ref[...] = lax.dot_general(
        x_ref[...], w_ref[...],
        dimension_numbers=(((1,), (1,)), ((), ())),
        preferred_element_type=jnp.float32,
    )


def _add_body(p_ref, o_ref):
    o_ref[...] = p_ref[0] + p_ref[1]


def _make_sc_aggregate(n_acc, n_edges, d):
    rows_per_tile = n_acc // N_SUBCORES
    edges_per_tile = n_edges // N_WORKERS
    n_chunks = edges_per_tile // CHUNK
    assert rows_per_tile * N_SUBCORES == n_acc and rows_per_tile % 8 == 0
    assert n_chunks * CHUNK == edges_per_tile

    mesh = plsc.VectorSubcoreMesh(core_axis_name="c", subcore_axis_name="s")

    @functools.partial(
        pl.kernel,
        out_type=jax.ShapeDtypeStruct((N_CORES, n_acc, d), jnp.float32),
        mesh=mesh,
        scratch_types=[
            pltpu.VMEM((CHUNK,), jnp.int32),
            pltpu.VMEM((CHUNK,), jnp.int32),
            pltpu.VMEM((CHUNK, d), jnp.float32),
            pltpu.VMEM_SHARED((n_acc, d), jnp.float32),
            pltpu.SemaphoreType.DMA,
        ],
    )
    def sc_aggregate(h_hbm, src_hbm, dst_hbm, zeros_hbm, out_hbm,
                     src_v, dst_v, rows_v, acc, sem):
        c = lax.axis_index("c")
        s = lax.axis_index("s")
        r0 = s * rows_per_tile
        pltpu.sync_copy(zeros_hbm.at[pl.ds(r0, rows_per_tile)],
                        acc.at[pl.ds(r0, rows_per_tile)])
        plsc.subcore_barrier()

        base = (c * N_SUBCORES + s) * edges_per_tile

        def body(i, _):
            eb = base + i * CHUNK
            pltpu.sync_copy(src_hbm.at[pl.ds(eb, CHUNK)], src_v)
            pltpu.sync_copy(dst_hbm.at[pl.ds(eb, CHUNK)], dst_v)
            pltpu.async_copy(h_hbm.at[src_v], rows_v, sem).wait()
            pltpu.sync_copy(rows_v, acc.at[dst_v], add=True)
            return ()

        lax.fori_loop(0, n_chunks, body, ())
        plsc.subcore_barrier()
        pltpu.sync_copy(acc.at[pl.ds(r0, rows_per_tile)],
                        out_hbm.at[c, pl.ds(r0, rows_per_tile)])

    return sc_aggregate


def kernel(x, edge_index, weight):
    n_nodes, feat = x.shape
    embed = weight.shape[0]
    n_edges = edge_index.shape[1]

    bm = 2000
    h = pl.pallas_call(
        _gemm_body,
        grid=(n_nodes // bm,),
        in_specs=[
            pl.BlockSpec((bm, feat), lambda i: (i, 0)),
            pl.BlockSpec((embed, feat), lambda i: (0, 0)),
        ],
        out_specs=pl.BlockSpec((bm, embed), lambda i: (i, 0)),
        out_shape=jax.ShapeDtypeStruct((n_nodes, embed), jnp.float32),
    )(x, weight)

    src = edge_index[0]
    dst = edge_index[1]
    n_acc = ((n_nodes + 8 * N_SUBCORES - 1) // (8 * N_SUBCORES)) * 8 * N_SUBCORES
    zeros = jnp.zeros((n_acc, embed), jnp.float32)
    partials = _make_sc_aggregate(n_acc, n_edges, embed)(h, src, dst, zeros)

    out = pl.pallas_call(
        _add_body,
        grid=(n_nodes // bm,),
        in_specs=[pl.BlockSpec((N_CORES, bm, embed), lambda i: (0, i, 0))],
        out_specs=pl.BlockSpec((bm, embed), lambda i: (i, 0)),
        out_shape=jax.ShapeDtypeStruct((n_nodes, embed), jnp.float32),
    )(partials)
    return out


# speedup vs baseline: 9.6374x; 1.7761x over previous
import functools

import jax
import jax.numpy as jnp
from jax import lax
from jax.experimental import pallas as pl
from jax.experimental.pallas import tpu as pltpu
from jax.experimental.pallas import tpu_sc as plsc

N_CORES = 2
N_SUBCORES = 16
N_WORKERS = N_CORES * N_SUBCORES
CHUNK = 80


def _gemm_body(x_ref, w_ref, o_ref):
    o_ref[...] = lax.dot_general(
        x_ref[...], w_ref[...],
        dimension_numbers=(((1,), (1,)), ((), ())),
        preferred_element_type=jnp.float32,
    )


def _add_body(p_ref, o_ref):
    o_ref[...] = p_ref[0] + p_ref[1]


def _make_sc_aggregate(n_acc, n_edges, d):
    rows_per_tile = n_acc // N_SUBCORES
    edges_per_tile = n_edges // N_WORKERS
    n_chunks = edges_per_tile // CHUNK
    assert rows_per_tile * N_SUBCORES == n_acc and rows_per_tile % 8 == 0
    assert n_chunks * CHUNK == edges_per_tile

    mesh = plsc.VectorSubcoreMesh(core_axis_name="c", subcore_axis_name="s")
    assert n_chunks % 2 == 1

    @functools.partial(
        pl.kernel,
        out_type=jax.ShapeDtypeStruct((N_CORES, n_acc, d), jnp.float32),
        mesh=mesh,
        scratch_types=[
            pltpu.VMEM((edges_per_tile,), jnp.int32),
            pltpu.VMEM((edges_per_tile,), jnp.int32),
            pltpu.VMEM((CHUNK,), jnp.int32),
            pltpu.VMEM((CHUNK,), jnp.int32),
            pltpu.VMEM((CHUNK,), jnp.int32),
            pltpu.VMEM((CHUNK,), jnp.int32),
            pltpu.VMEM((CHUNK, d), jnp.float32),
            pltpu.VMEM((CHUNK, d), jnp.float32),
            pltpu.VMEM_SHARED((n_acc, d), jnp.float32),
            pltpu.SemaphoreType.DMA,
            pltpu.SemaphoreType.DMA,
        ],
    )
    def sc_aggregate(h_hbm, src_hbm, dst_hbm, zeros_hbm, out_hbm,
                     srcb, dstb, src_v0, src_v1, dst_v0, dst_v1,
                     rows0, rows1, acc, sem0, sem1):
        c = lax.axis_index("c")
        s = lax.axis_index("s")
        src_v = (src_v0, src_v1)
        dst_v = (dst_v0, dst_v1)
        rows = (rows0, rows1)
        sems = (sem0, sem1)
        base = (c * N_SUBCORES + s) * edges_per_tile

        cp_src = pltpu.async_copy(src_hbm.at[pl.ds(base, edges_per_tile)],
                                  srcb, sem0)
        cp_dst = pltpu.async_copy(dst_hbm.at[pl.ds(base, edges_per_tile)],
                                  dstb, sem1)
        r0 = s * rows_per_tile
        pltpu.sync_copy(zeros_hbm.at[pl.ds(r0, rows_per_tile)],
                        acc.at[pl.ds(r0, rows_per_tile)])
        cp_src.wait()
        cp_dst.wait()

        def prep_idx(j, b):
            for t in range(CHUNK // 16):
                sl = pl.ds(t * 16, 16)
                src_v[b][sl] = srcb[pl.ds(j * CHUNK + t * 16, 16)]
                dst_v[b][sl] = dstb[pl.ds(j * CHUNK + t * 16, 16)]

        def start_gather(b):
            return pltpu.async_copy(h_hbm.at[src_v[b]], rows[b], sems[b])

        prep_idx(0, 0)
        start_gather(0)
        plsc.subcore_barrier()

        def step(j, b, has_next):
            if has_next:
                prep_idx(j + 1, 1 - b)
            pltpu.make_async_copy(h_hbm.at[src_v[b]], rows[b], sems[b]).wait()
            if has_next:
                start_gather(1 - b)
            pltpu.sync_copy(rows[b], acc.at[dst_v[b]], add=True)

        def body(k, _):
            step(2 * k, 0, True)
            step(2 * k + 1, 1, True)
            return ()

        lax.fori_loop(0, (n_chunks - 1) // 2, body, ())
        step(n_chunks - 1, 0, False)
        plsc.subcore_barrier()
        pltpu.sync_copy(acc.at[pl.ds(r0, rows_per_tile)],
                        out_hbm.at[c, pl.ds(r0, rows_per_tile)])

    return sc_aggregate


def kernel(x, edge_index, weight):
    n_nodes, feat = x.shape
    embed = weight.shape[0]
    n_edges = edge_index.shape[1]

    bm = 2000
    h = pl.pallas_call(
        _gemm_body,
        grid=(n_nodes // bm,),
        in_specs=[
            pl.BlockSpec((bm, feat), lambda i: (i, 0)),
            pl.BlockSpec((embed, feat), lambda i: (0, 0)),
        ],
        out_specs=pl.BlockSpec((bm, embed), lambda i: (i, 0)),
        out_shape=jax.ShapeDtypeStruct((n_nodes, embed), jnp.float32),
    )(x, weight)

    src = edge_index[0]
    dst = edge_index[1]
    n_acc = ((n_nodes + 8 * N_SUBCORES - 1) // (8 * N_SUBCORES)) * 8 * N_SUBCORES
    zeros = jnp.zeros((n_acc, embed), jnp.float32)
    partials = _make_sc_aggregate(n_acc, n_edges, embed)(h, src, dst, zeros)

    out = pl.pallas_call(
        _add_body,
        grid=(n_nodes // bm,),
        in_specs=[pl.BlockSpec((N_CORES, bm, embed), lambda i: (0, i, 0))],
        out_specs=pl.BlockSpec((bm, embed), lambda i: (i, 0)),
        out_shape=jax.ShapeDtypeStruct((n_nodes, embed), jnp.float32),
    )(partials)
    return out


# speedup vs baseline: 9.7877x; 1.0156x over previous
import functools

import jax
import jax.numpy as jnp
from jax import lax
from jax.experimental import pallas as pl
from jax.experimental.pallas import tpu as pltpu
from jax.experimental.pallas import tpu_sc as plsc

N_CORES = 2
N_SUBCORES = 16
N_WORKERS = N_CORES * N_SUBCORES
CHUNK = 80


def _gemm_body(x_ref, w_ref, o_ref):
    o_ref[...] = lax.dot_general(
        x_ref[...], w_ref[...],
        dimension_numbers=(((1,), (1,)), ((), ())),
        preferred_element_type=jnp.float32,
    )


def _add_body(p_ref, o_ref):
    o_ref[...] = p_ref[0] + p_ref[1]


def _make_sc_aggregate(n_acc, n_edges, d):
    rows_per_tile = n_acc // N_SUBCORES
    edges_per_tile = n_edges // N_WORKERS
    n_chunks = edges_per_tile // CHUNK
    assert rows_per_tile * N_SUBCORES == n_acc and rows_per_tile % 8 == 0
    assert n_chunks * CHUNK == edges_per_tile

    mesh = plsc.VectorSubcoreMesh(core_axis_name="c", subcore_axis_name="s")
    NBUF = 3
    ZROWS = 64
    assert rows_per_tile % ZROWS == 0
    assert n_chunks >= 6 and (n_chunks - 5) % NBUF == 0

    @functools.partial(
        pl.kernel,
        out_type=jax.ShapeDtypeStruct((N_CORES, n_acc, d), jnp.float32),
        mesh=mesh,
        scratch_types=[
            [pltpu.VMEM((CHUNK,), jnp.int32)] * NBUF,
            [pltpu.VMEM((CHUNK,), jnp.int32)] * NBUF,
            [pltpu.VMEM((CHUNK, d), jnp.float32)] * NBUF,
            pltpu.VMEM((ZROWS, d), jnp.float32),
            pltpu.VMEM_SHARED((n_acc, d), jnp.float32),
            [pltpu.SemaphoreType.DMA] * NBUF,
            [pltpu.SemaphoreType.DMA] * NBUF,
            [pltpu.SemaphoreType.DMA] * NBUF,
        ],
    )
    def sc_aggregate(h_hbm, src_hbm, dst_hbm, out_hbm,
                     src_v, dst_v, rows, zbuf, acc, isem, gsem, ssem):
        c = lax.axis_index("c")
        s = lax.axis_index("s")
        base = (c * N_SUBCORES + s) * edges_per_tile
        r0 = s * rows_per_tile

        def start_idx(j, b):
            eb = base + j * CHUNK
            pltpu.async_copy(src_hbm.at[pl.ds(eb, CHUNK)], src_v[b], isem[b])
            pltpu.async_copy(dst_hbm.at[pl.ds(eb, CHUNK)], dst_v[b], isem[b])

        def wait_idx(j, b):
            eb = base + j * CHUNK
            pltpu.make_async_copy(src_hbm.at[pl.ds(eb, CHUNK)], src_v[b],
                                  isem[b]).wait()
            pltpu.make_async_copy(dst_hbm.at[pl.ds(eb, CHUNK)], dst_v[b],
                                  isem[b]).wait()

        def start_gather(b):
            pltpu.async_copy(h_hbm.at[src_v[b]], rows[b], gsem[b])

        def wait_gather(b):
            pltpu.make_async_copy(h_hbm.at[src_v[b]], rows[b], gsem[b]).wait()

        def start_scatter(b):
            pltpu.async_copy(rows[b], acc.at[dst_v[b]], ssem[b], add=True)

        def wait_scatter(b):
            pltpu.make_async_copy(rows[b], acc.at[dst_v[b]], ssem[b]).wait()

        start_idx(0, 0)
        start_idx(1, 1)
        zero16 = jnp.zeros((16,), jnp.float32)

        def zrow(i, _):
            for t in range(d // 16):
                zbuf[i, pl.ds(t * 16, 16)] = zero16
            return ()

        lax.fori_loop(0, ZROWS, zrow, ())
        for m in range(rows_per_tile // ZROWS):
            pltpu.sync_copy(zbuf, acc.at[pl.ds(r0 + m * ZROWS, ZROWS)])
        wait_idx(0, 0)
        start_gather(0)
        plsc.subcore_barrier()

        def step(j, b, wait_prev_scatter, prep_idx2, start_next_gather):
            b1 = (b + 1) % NBUF
            b2 = (b + 2) % NBUF
            wait_gather(b)
            start_scatter(b)
            if wait_prev_scatter:
                wait_scatter(b2)
            if prep_idx2:
                start_idx(j + 2, b2)
            if start_next_gather:
                wait_idx(j + 1, b1)
                start_gather(b1)

        step(0, 0, False, True, True)

        def body(k, _):
            j = NBUF * k + 1
            step(j, 1, True, True, True)
            step(j + 1, 2, True, True, True)
            step(j + 2, 0, True, True, True)
            return ()

        lax.fori_loop(0, (n_chunks - 5) // NBUF, body, ())
        step(n_chunks - 4, (n_chunks - 4) % NBUF, True, True, True)
        step(n_chunks - 3, (n_chunks - 3) % NBUF, True, True, True)
        step(n_chunks - 2, (n_chunks - 2) % NBUF, True, False, True)
        step(n_chunks - 1, (n_chunks - 1) % NBUF, True, False, False)
        wait_scatter((n_chunks - 1) % NBUF)
        plsc.subcore_barrier()
        pltpu.sync_copy(acc.at[pl.ds(r0, rows_per_tile)],
                        out_hbm.at[c, pl.ds(r0, rows_per_tile)])

    return sc_aggregate


def kernel(x, edge_index, weight):
    n_nodes, feat = x.shape
    embed = weight.shape[0]
    n_edges = edge_index.shape[1]

    bm = 2000
    h = pl.pallas_call(
        _gemm_body,
        grid=(n_nodes // bm,),
        in_specs=[
            pl.BlockSpec((bm, feat), lambda i: (i, 0)),
            pl.BlockSpec((embed, feat), lambda i: (0, 0)),
        ],
        out_specs=pl.BlockSpec((bm, embed), lambda i: (i, 0)),
        out_shape=jax.ShapeDtypeStruct((n_nodes, embed), jnp.float32),
    )(x, weight)

    src = edge_index[0]
    dst = edge_index[1]
    pad = 64 * N_SUBCORES
    n_acc = ((n_nodes + pad - 1) // pad) * pad
    partials = _make_sc_aggregate(n_acc, n_edges, embed)(h, src, dst)

    out = pl.pallas_call(
        _add_body,
        grid=(n_nodes // bm,),
        in_specs=[pl.BlockSpec((N_CORES, bm, embed), lambda i: (0, i, 0))],
        out_specs=pl.BlockSpec((bm, embed), lambda i: (i, 0)),
        out_shape=jax.ShapeDtypeStruct((n_nodes, embed), jnp.float32),
    )(partials)
    return out


# speedup vs baseline: 13.6709x; 1.3968x over previous
import functools

import jax
import jax.numpy as jnp
from jax import lax
from jax.experimental import pallas as pl
from jax.experimental.pallas import tpu as pltpu
from jax.experimental.pallas import tpu_sc as plsc

N_CORES = 2
N_SUBCORES = 16
N_WORKERS = N_CORES * N_SUBCORES
CHUNK = 80


def _gemm_body(x_ref, w_ref, o_ref):
    o_ref[...] = lax.dot_general(
        x_ref[...], w_ref[...],
        dimension_numbers=(((1,), (1,)), ((), ())),
        preferred_element_type=jnp.float32,
    )


def _add_body(p_ref, o_ref):
    o_ref[...] = p_ref[0] + p_ref[1]


def _make_sc_aggregate(n_acc, n_edges, d):
    rows_per_tile = n_acc // N_SUBCORES
    edges_per_tile = n_edges // N_WORKERS
    n_chunks = edges_per_tile // CHUNK
    assert rows_per_tile * N_SUBCORES == n_acc and rows_per_tile % 8 == 0
    assert n_chunks * CHUNK == edges_per_tile

    mesh = plsc.VectorSubcoreMesh(core_axis_name="c", subcore_axis_name="s")
    NBUF = 4
    ZROWS = 32
    assert rows_per_tile % ZROWS == 0
    assert n_chunks >= 6 and (n_chunks - 5) % NBUF == 0

    @functools.partial(
        pl.kernel,
        out_type=jax.ShapeDtypeStruct((N_CORES, n_acc, d), jnp.float32),
        mesh=mesh,
        scratch_types=[
            [pltpu.VMEM((CHUNK,), jnp.int32)] * NBUF,
            [pltpu.VMEM((CHUNK,), jnp.int32)] * NBUF,
            [pltpu.VMEM((CHUNK, d), jnp.float32)] * NBUF,
            pltpu.VMEM((ZROWS, d), jnp.float32),
            pltpu.VMEM_SHARED((n_acc, d), jnp.float32),
            [pltpu.SemaphoreType.DMA] * NBUF,
            [pltpu.SemaphoreType.DMA] * NBUF,
            [pltpu.SemaphoreType.DMA] * NBUF,
        ],
    )
    def sc_aggregate(h_hbm, src_hbm, dst_hbm, out_hbm,
                     src_v, dst_v, rows, zbuf, acc, isem, gsem, ssem):
        c = lax.axis_index("c")
        s = lax.axis_index("s")
        base = (c * N_SUBCORES + s) * edges_per_tile
        r0 = s * rows_per_tile

        def start_idx(j, b):
            eb = base + j * CHUNK
            pltpu.async_copy(src_hbm.at[pl.ds(eb, CHUNK)], src_v[b], isem[b])
            pltpu.async_copy(dst_hbm.at[pl.ds(eb, CHUNK)], dst_v[b], isem[b])

        def wait_idx(j, b):
            eb = base + j * CHUNK
            pltpu.make_async_copy(src_hbm.at[pl.ds(eb, CHUNK)], src_v[b],
                                  isem[b]).wait()
            pltpu.make_async_copy(dst_hbm.at[pl.ds(eb, CHUNK)], dst_v[b],
                                  isem[b]).wait()

        def start_gather(b):
            pltpu.async_copy(h_hbm.at[src_v[b]], rows[b], gsem[b])

        def wait_gather(b):
            pltpu.make_async_copy(h_hbm.at[src_v[b]], rows[b], gsem[b]).wait()

        def start_scatter(b):
            pltpu.async_copy(rows[b], acc.at[dst_v[b]], ssem[b], add=True)

        def wait_scatter(b):
            pltpu.make_async_copy(rows[b], acc.at[dst_v[b]], ssem[b]).wait()

        start_idx(0, 0)
        start_idx(1, 1)
        start_idx(2, 2)
        zero16 = jnp.zeros((16,), jnp.float32)

        def zrow(i, _):
            for t in range(d // 16):
                zbuf[i, pl.ds(t * 16, 16)] = zero16
            return ()

        lax.fori_loop(0, ZROWS, zrow, ())
        for m in range(rows_per_tile // ZROWS):
            pltpu.sync_copy(zbuf, acc.at[pl.ds(r0 + m * ZROWS, ZROWS)])
        wait_idx(0, 0)
        start_gather(0)
        wait_idx(1, 1)
        start_gather(1)
        plsc.subcore_barrier()

        def step(j, b, wait_prev_scatter, prep_idx3, start_gather2):
            b2 = (b + 2) % NBUF
            b3 = (b + 3) % NBUF
            wait_gather(b)
            start_scatter(b)
            if wait_prev_scatter:
                wait_scatter(b3)
            if prep_idx3:
                start_idx(j + 3, b3)
            if start_gather2:
                wait_idx(j + 2, b2)
                start_gather(b2)

        step(0, 0, False, True, True)
        step(1, 1, True, True, True)

        def body(k, _):
            j = NBUF * k + 2
            step(j, 2, True, True, True)
            step(j + 1, 3, True, True, True)
            step(j + 2, 0, True, True, True)
            step(j + 3, 1, True, True, True)
            return ()

        lax.fori_loop(0, (n_chunks - 5) // NBUF, body, ())
        step(n_chunks - 3, (n_chunks - 3) % NBUF, True, False, True)
        step(n_chunks - 2, (n_chunks - 2) % NBUF, True, False, False)
        step(n_chunks - 1, (n_chunks - 1) % NBUF, True, False, False)
        wait_scatter((n_chunks - 1) % NBUF)
        plsc.subcore_barrier()
        pltpu.sync_copy(acc.at[pl.ds(r0, rows_per_tile)],
                        out_hbm.at[c, pl.ds(r0, rows_per_tile)])

    return sc_aggregate


def kernel(x, edge_index, weight):
    n_nodes, feat = x.shape
    embed = weight.shape[0]
    n_edges = edge_index.shape[1]

    bm = 2000
    h = pl.pallas_call(
        _gemm_body,
        grid=(n_nodes // bm,),
        in_specs=[
            pl.BlockSpec((bm, feat), lambda i: (i, 0)),
            pl.BlockSpec((embed, feat), lambda i: (0, 0)),
        ],
        out_specs=pl.BlockSpec((bm, embed), lambda i: (i, 0)),
        out_shape=jax.ShapeDtypeStruct((n_nodes, embed), jnp.float32),
    )(x, weight)

    src = edge_index[0]
    dst = edge_index[1]
    pad = 64 * N_SUBCORES
    n_acc = ((n_nodes + pad - 1) // pad) * pad
    partials = _make_sc_aggregate(n_acc, n_edges, embed)(h, src, dst)

    out = pl.pallas_call(
        _add_body,
        grid=(n_nodes // bm,),
        in_specs=[pl.BlockSpec((N_CORES, bm, embed), lambda i: (0, i, 0))],
        out_specs=pl.BlockSpec((bm, embed), lambda i: (i, 0)),
        out_shape=jax.ShapeDtypeStruct((n_nodes, embed), jnp.float32),
    )(partials)
    return out


# speedup vs baseline: 13.7925x; 1.0089x over previous
import functools

import jax
import jax.numpy as jnp
from jax import lax
from jax.experimental import pallas as pl
from jax.experimental.pallas import tpu as pltpu
from jax.experimental.pallas import tpu_sc as plsc

N_CORES = 2
N_SUBCORES = 16
N_WORKERS = N_CORES * N_SUBCORES
CHUNK = 40


def _gemm_body(x_ref, w_ref, o_ref):
    o_ref[...] = lax.dot_general(
        x_ref[...], w_ref[...],
        dimension_numbers=(((1,), (1,)), ((), ())),
        preferred_element_type=jnp.float32,
    )


def _add_body(p_ref, o_ref):
    o_ref[...] = p_ref[0] + p_ref[1]


def _make_sc_aggregate(n_acc, n_edges, d):
    rows_per_tile = n_acc // N_SUBCORES
    edges_per_tile = n_edges // N_WORKERS
    n_chunks = edges_per_tile // CHUNK
    assert rows_per_tile * N_SUBCORES == n_acc and rows_per_tile % 8 == 0
    assert n_chunks * CHUNK == edges_per_tile

    mesh = plsc.VectorSubcoreMesh(core_axis_name="c", subcore_axis_name="s")
    NBUF = 8
    ZROWS = 32
    assert rows_per_tile % ZROWS == 0
    PEEL = next(p for p in range(1, NBUF + 1)
                if (n_chunks - NBUF + 1 - p) % NBUF == 0)
    assert n_chunks >= PEEL + 2 * NBUF

    @functools.partial(
        pl.kernel,
        out_type=jax.ShapeDtypeStruct((N_CORES, n_acc, d), jnp.float32),
        mesh=mesh,
        scratch_types=[
            [pltpu.VMEM((CHUNK,), jnp.int32)] * NBUF,
            [pltpu.VMEM((CHUNK,), jnp.int32)] * NBUF,
            [pltpu.VMEM((CHUNK, d), jnp.float32)] * NBUF,
            pltpu.VMEM((ZROWS, d), jnp.float32),
            pltpu.VMEM_SHARED((n_acc, d), jnp.float32),
            [pltpu.SemaphoreType.DMA] * NBUF,
            [pltpu.SemaphoreType.DMA] * NBUF,
            [pltpu.SemaphoreType.DMA] * NBUF,
        ],
    )
    def sc_aggregate(h_hbm, src_hbm, dst_hbm, out_hbm,
                     src_v, dst_v, rows, zbuf, acc, isem, gsem, ssem):
        c = lax.axis_index("c")
        s = lax.axis_index("s")
        base = (c * N_SUBCORES + s) * edges_per_tile
        r0 = s * rows_per_tile

        def start_idx(j, b):
            eb = base + j * CHUNK
            pltpu.async_copy(src_hbm.at[pl.ds(eb, CHUNK)], src_v[b], isem[b])
            pltpu.async_copy(dst_hbm.at[pl.ds(eb, CHUNK)], dst_v[b], isem[b])

        def wait_idx(j, b):
            eb = base + j * CHUNK
            pltpu.make_async_copy(src_hbm.at[pl.ds(eb, CHUNK)], src_v[b],
                                  isem[b]).wait()
            pltpu.make_async_copy(dst_hbm.at[pl.ds(eb, CHUNK)], dst_v[b],
                                  isem[b]).wait()

        def start_gather(b):
            pltpu.async_copy(h_hbm.at[src_v[b]], rows[b], gsem[b])

        def wait_gather(b):
            pltpu.make_async_copy(h_hbm.at[src_v[b]], rows[b], gsem[b]).wait()

        def start_scatter(b):
            pltpu.async_copy(rows[b], acc.at[dst_v[b]], ssem[b], add=True)

        def wait_scatter(b):
            pltpu.make_async_copy(rows[b], acc.at[dst_v[b]], ssem[b]).wait()

        for b in range(NBUF - 1):
            start_idx(b, b)
        zero16 = jnp.zeros((16,), jnp.float32)

        def zrow(i, _):
            for t in range(d // 16):
                zbuf[i, pl.ds(t * 16, 16)] = zero16
            return ()

        lax.fori_loop(0, ZROWS, zrow, ())
        for m in range(rows_per_tile // ZROWS):
            pltpu.sync_copy(zbuf, acc.at[pl.ds(r0 + m * ZROWS, ZROWS)])
        for b in range(NBUF - 2):
            wait_idx(b, b)
            start_gather(b)
        plsc.subcore_barrier()

        def step(j, b, wait_prev_scatter, do_idx, do_gather):
            bm1 = (b + NBUF - 1) % NBUF
            bm2 = (b + NBUF - 2) % NBUF
            wait_gather(b)
            start_scatter(b)
            if wait_prev_scatter:
                wait_scatter(bm1)
            if do_idx:
                start_idx(j + NBUF - 1, bm1)
            if do_gather:
                wait_idx(j + NBUF - 2, bm2)
                start_gather(bm2)

        for j in range(PEEL):
            step(j, j % NBUF, j > 0, True, True)

        def body(k, _):
            j0 = NBUF * k + PEEL
            for t in range(NBUF):
                step(j0 + t, (PEEL + t) % NBUF, True, True, True)
            return ()

        n_full = n_chunks - NBUF + 1 - PEEL
        lax.fori_loop(0, n_full // NBUF, body, ())
        j1 = n_chunks - NBUF + 1
        step(j1, j1 % NBUF, True, False, True)
        for j in range(j1 + 1, n_chunks):
            step(j, j % NBUF, True, False, False)
        wait_scatter((n_chunks - 1) % NBUF)
        plsc.subcore_barrier()
        pltpu.sync_copy(acc.at[pl.ds(r0, rows_per_tile)],
                        out_hbm.at[c, pl.ds(r0, rows_per_tile)])

    return sc_aggregate


def kernel(x, edge_index, weight):
    n_nodes, feat = x.shape
    embed = weight.shape[0]
    n_edges = edge_index.shape[1]

    bm = 2000
    h = pl.pallas_call(
        _gemm_body,
        grid=(n_nodes // bm,),
        in_specs=[
            pl.BlockSpec((bm, feat), lambda i: (i, 0)),
            pl.BlockSpec((embed, feat), lambda i: (0, 0)),
        ],
        out_specs=pl.BlockSpec((bm, embed), lambda i: (i, 0)),
        out_shape=jax.ShapeDtypeStruct((n_nodes, embed), jnp.float32),
    )(x, weight)

    src = edge_index[0]
    dst = edge_index[1]
    pad = 64 * N_SUBCORES
    n_acc = ((n_nodes + pad - 1) // pad) * pad
    partials = _make_sc_aggregate(n_acc, n_edges, embed)(h, src, dst)

    out = pl.pallas_call(
        _add_body,
        grid=(n_nodes // bm,),
        in_specs=[pl.BlockSpec((N_CORES, bm, embed), lambda i: (0, i, 0))],
        out_specs=pl.BlockSpec((bm, embed), lambda i: (i, 0)),
        out_shape=jax.ShapeDtypeStruct((n_nodes, embed), jnp.float32),
    )(partials)
    return out


# speedup vs baseline: 14.8977x; 1.0801x over previous
import functools

import jax
import jax.numpy as jnp
from jax import lax
from jax.experimental import pallas as pl
from jax.experimental.pallas import tpu as pltpu
from jax.experimental.pallas import tpu_sc as plsc

N_CORES = 2
N_SUBCORES = 16
N_WORKERS = N_CORES * N_SUBCORES
CHUNK = 40


def _gemm_body(x_ref, w_ref, o_ref):
    o_ref[...] = lax.dot_general(
        x_ref[...], w_ref[...],
        dimension_numbers=(((1,), (1,)), ((), ())),
        preferred_element_type=jnp.float32,
    )


def _add_body(p_ref, o_ref):
    o_ref[...] = p_ref[0] + p_ref[1]


def _make_sc_aggregate(n_acc, n_edges, d):
    rows_per_tile = n_acc // N_SUBCORES
    edges_per_tile = n_edges // N_WORKERS
    n_chunks = edges_per_tile // CHUNK
    assert rows_per_tile * N_SUBCORES == n_acc and rows_per_tile % 8 == 0
    assert n_chunks * CHUNK == edges_per_tile

    mesh = plsc.VectorSubcoreMesh(core_axis_name="c", subcore_axis_name="s")
    NBUF = 8
    ZROWS = 32
    assert rows_per_tile % ZROWS == 0
    PEEL = next(p for p in range(1, NBUF + 1)
                if (n_chunks - NBUF + 1 - p) % NBUF == 0)
    assert n_chunks >= PEEL + 2 * NBUF

    @functools.partial(
        pl.kernel,
        out_type=jax.ShapeDtypeStruct((N_CORES, n_acc, d), jnp.float32),
        mesh=mesh,
        scratch_types=[
            [pltpu.VMEM((CHUNK,), jnp.int32)] * NBUF,
            [pltpu.VMEM((CHUNK,), jnp.int32)] * NBUF,
            [pltpu.VMEM((CHUNK, d), jnp.float32)] * NBUF,
            pltpu.VMEM((ZROWS, d), jnp.float32),
            pltpu.VMEM_SHARED((n_acc, d), jnp.float32),
            [pltpu.SemaphoreType.DMA] * NBUF,
            [pltpu.SemaphoreType.DMA] * NBUF,
            [pltpu.SemaphoreType.DMA] * NBUF,
        ],
    )
    def sc_aggregate(h_hbm, edge_hbm, out_hbm,
                     src_v, dst_v, rows, zbuf, acc, isem, gsem, ssem):
        c = lax.axis_index("c")
        s = lax.axis_index("s")
        base = (c * N_SUBCORES + s) * edges_per_tile
        r0 = s * rows_per_tile

        def start_idx(j, b):
            eb = base + j * CHUNK
            pltpu.async_copy(edge_hbm.at[pl.ds(eb, CHUNK)], src_v[b],
                             isem[b])
            pltpu.async_copy(edge_hbm.at[pl.ds(n_edges + eb, CHUNK)],
                             dst_v[b], isem[b])

        def wait_idx(j, b):
            eb = base + j * CHUNK
            pltpu.make_async_copy(edge_hbm.at[pl.ds(eb, CHUNK)], src_v[b],
                                  isem[b]).wait()
            pltpu.make_async_copy(edge_hbm.at[pl.ds(n_edges + eb, CHUNK)],
                                  dst_v[b], isem[b]).wait()

        def start_gather(b):
            pltpu.async_copy(h_hbm.at[src_v[b]], rows[b], gsem[b])

        def wait_gather(b):
            pltpu.make_async_copy(h_hbm.at[src_v[b]], rows[b], gsem[b]).wait()

        def start_scatter(b):
            pltpu.async_copy(rows[b], acc.at[dst_v[b]], ssem[b], add=True)

        def wait_scatter(b):
            pltpu.make_async_copy(rows[b], acc.at[dst_v[b]], ssem[b]).wait()

        for b in range(NBUF - 1):
            start_idx(b, b)
        zero16 = jnp.zeros((16,), jnp.float32)

        def zrow(i, _):
            for t in range(d // 16):
                zbuf[i, pl.ds(t * 16, 16)] = zero16
            return ()

        lax.fori_loop(0, ZROWS, zrow, ())
        for m in range(rows_per_tile // ZROWS):
            pltpu.sync_copy(zbuf, acc.at[pl.ds(r0 + m * ZROWS, ZROWS)])
        for b in range(NBUF - 2):
            wait_idx(b, b)
            start_gather(b)
        plsc.subcore_barrier()

        def step(j, b, wait_prev_scatter, do_idx, do_gather):
            bm1 = (b + NBUF - 1) % NBUF
            bm2 = (b + NBUF - 2) % NBUF
            wait_gather(b)
            start_scatter(b)
            if wait_prev_scatter:
                wait_scatter(bm1)
            if do_idx:
                start_idx(j + NBUF - 1, bm1)
            if do_gather:
                wait_idx(j + NBUF - 2, bm2)
                start_gather(bm2)

        for j in range(PEEL):
            step(j, j % NBUF, j > 0, True, True)

        def body(k, _):
            j0 = NBUF * k + PEEL
            for t in range(NBUF):
                step(j0 + t, (PEEL + t) % NBUF, True, True, True)
            return ()

        n_full = n_chunks - NBUF + 1 - PEEL
        lax.fori_loop(0, n_full // NBUF, body, ())
        j1 = n_chunks - NBUF + 1
        step(j1, j1 % NBUF, True, False, True)
        for j in range(j1 + 1, n_chunks):
            step(j, j % NBUF, True, False, False)
        wait_scatter((n_chunks - 1) % NBUF)
        plsc.subcore_barrier()
        pltpu.sync_copy(acc.at[pl.ds(r0, rows_per_tile)],
                        out_hbm.at[c, pl.ds(r0, rows_per_tile)])

    return sc_aggregate


def kernel(x, edge_index, weight):
    n_nodes, feat = x.shape
    embed = weight.shape[0]
    n_edges = edge_index.shape[1]

    bm = 2000
    h = pl.pallas_call(
        _gemm_body,
        grid=(n_nodes // bm,),
        in_specs=[
            pl.BlockSpec((bm, feat), lambda i: (i, 0)),
            pl.BlockSpec((embed, feat), lambda i: (0, 0)),
        ],
        out_specs=pl.BlockSpec((bm, embed), lambda i: (i, 0)),
        out_shape=jax.ShapeDtypeStruct((n_nodes, embed), jnp.float32),
    )(x, weight)

    pad = 64 * N_SUBCORES
    n_acc = ((n_nodes + pad - 1) // pad) * pad
    edge_flat = edge_index.reshape(2 * n_edges)
    partials = _make_sc_aggregate(n_acc, n_edges, embed)(h, edge_flat)

    out = pl.pallas_call(
        _add_body,
        grid=(n_nodes // bm,),
        in_specs=[pl.BlockSpec((N_CORES, bm, embed), lambda i: (0, i, 0))],
        out_specs=pl.BlockSpec((bm, embed), lambda i: (i, 0)),
        out_shape=jax.ShapeDtypeStruct((n_nodes, embed), jnp.float32),
    )(partials)
    return out


# speedup vs baseline: 15.0036x; 1.0071x over previous
import functools

import jax
import jax.numpy as jnp
from jax import lax
from jax.experimental import pallas as pl
from jax.experimental.pallas import tpu as pltpu
from jax.experimental.pallas import tpu_sc as plsc

N_CORES = 2
N_SUBCORES = 16
N_WORKERS = N_CORES * N_SUBCORES
CHUNK = 40


def _gemm_body(x_ref, w_ref, o_ref):
    o_ref[...] = lax.dot_general(
        x_ref[...], w_ref[...],
        dimension_numbers=(((1,), (1,)), ((), ())),
        preferred_element_type=jnp.float32,
    )


def _add_body(p_ref, o_ref):
    o_ref[...] = p_ref[0] + p_ref[1]


def _make_sc_aggregate(n_acc, n_edges, d):
    rows_per_tile = n_acc // N_SUBCORES
    edges_per_tile = n_edges // N_WORKERS
    n_chunks = edges_per_tile // CHUNK
    assert rows_per_tile * N_SUBCORES == n_acc and rows_per_tile % 8 == 0
    assert n_chunks * CHUNK == edges_per_tile

    mesh = plsc.VectorSubcoreMesh(core_axis_name="c", subcore_axis_name="s")
    NBUF = 8
    ZROWS = 32
    assert rows_per_tile % ZROWS == 0
    PEEL = next(p for p in range(1, NBUF + 1)
                if (n_chunks - NBUF + 1 - p) % NBUF == 0)
    assert n_chunks >= PEEL + 2 * NBUF

    @functools.partial(
        pl.kernel,
        out_type=jax.ShapeDtypeStruct((N_CORES, n_acc, d), jnp.float32),
        mesh=mesh,
        scratch_types=[
            [pltpu.VMEM((CHUNK,), jnp.int32)] * NBUF,
            [pltpu.VMEM((CHUNK,), jnp.int32)] * NBUF,
            [pltpu.VMEM((CHUNK, d), jnp.float32)] * NBUF,
            pltpu.VMEM((ZROWS, d), jnp.float32),
            pltpu.VMEM_SHARED((n_acc, d), jnp.float32),
            [pltpu.SemaphoreType.DMA] * NBUF,
            [pltpu.SemaphoreType.DMA] * NBUF,
            [pltpu.SemaphoreType.DMA] * NBUF,
        ],
    )
    def sc_aggregate(h_hbm, edge_hbm, out_hbm,
                     src_v, dst_v, rows, zbuf, acc, isem, gsem, ssem):
        c = lax.axis_index("c")
        s = lax.axis_index("s")
        base = (c * N_SUBCORES + s) * edges_per_tile
        r0 = s * rows_per_tile

        def start_idx(j, b):
            eb = base + j * CHUNK
            pltpu.async_copy(edge_hbm.at[pl.ds(eb, CHUNK)], src_v[b],
                             isem[b])
            pltpu.async_copy(edge_hbm.at[pl.ds(n_edges + eb, CHUNK)],
                             dst_v[b], isem[b])

        def wait_idx(j, b):
            eb = base + j * CHUNK
            pltpu.make_async_copy(edge_hbm.at[pl.ds(eb, CHUNK)], src_v[b],
                                  isem[b]).wait()
            pltpu.make_async_copy(edge_hbm.at[pl.ds(n_edges + eb, CHUNK)],
                                  dst_v[b], isem[b]).wait()

        def start_gather(b):
            pltpu.async_copy(h_hbm.at[src_v[b]], rows[b], gsem[b])

        def wait_gather(b):
            pltpu.make_async_copy(h_hbm.at[src_v[b]], rows[b], gsem[b]).wait()

        def start_scatter(b):
            pltpu.async_copy(rows[b], acc.at[dst_v[b]], ssem[b], add=True)

        def wait_scatter(b):
            pltpu.make_async_copy(rows[b], acc.at[dst_v[b]], ssem[b]).wait()

        for b in range(NBUF - 1):
            start_idx(b, b)
        zero16 = jnp.zeros((16,), jnp.float32)

        def zrow(i, _):
            for t in range(d // 16):
                zbuf[i, pl.ds(t * 16, 16)] = zero16
            return ()

        lax.fori_loop(0, ZROWS, zrow, ())
        zsem = isem[NBUF - 1]
        for m in range(rows_per_tile // ZROWS):
            pltpu.async_copy(zbuf, acc.at[pl.ds(r0 + m * ZROWS, ZROWS)], zsem)
        for m in range(rows_per_tile // ZROWS):
            pltpu.make_async_copy(zbuf, acc.at[pl.ds(r0 + m * ZROWS, ZROWS)],
                                  zsem).wait()
        for b in range(NBUF - 2):
            wait_idx(b, b)
            start_gather(b)
        plsc.subcore_barrier()

        def step(j, b, wait_prev_scatter, do_idx, do_gather):
            bm1 = (b + NBUF - 1) % NBUF
            bm2 = (b + NBUF - 2) % NBUF
            wait_gather(b)
            start_scatter(b)
            if wait_prev_scatter:
                wait_scatter(bm1)
            if do_idx:
                start_idx(j + NBUF - 1, bm1)
            if do_gather:
                wait_idx(j + NBUF - 2, bm2)
                start_gather(bm2)

        for j in range(PEEL):
            step(j, j % NBUF, j > 0, True, True)

        def body(k, _):
            j0 = NBUF * k + PEEL
            for t in range(NBUF):
                step(j0 + t, (PEEL + t) % NBUF, True, True, True)
            return ()

        n_full = n_chunks - NBUF + 1 - PEEL
        lax.fori_loop(0, n_full // NBUF, body, ())
        j1 = n_chunks - NBUF + 1
        step(j1, j1 % NBUF, True, False, True)
        for j in range(j1 + 1, n_chunks):
            step(j, j % NBUF, True, False, False)
        wait_scatter((n_chunks - 1) % NBUF)
        plsc.subcore_barrier()
        pltpu.sync_copy(acc.at[pl.ds(r0, rows_per_tile)],
                        out_hbm.at[c, pl.ds(r0, rows_per_tile)])

    return sc_aggregate


def kernel(x, edge_index, weight):
    n_nodes, feat = x.shape
    embed = weight.shape[0]
    n_edges = edge_index.shape[1]

    bm = 2000
    h = pl.pallas_call(
        _gemm_body,
        grid=(n_nodes // bm,),
        in_specs=[
            pl.BlockSpec((bm, feat), lambda i: (i, 0)),
            pl.BlockSpec((embed, feat), lambda i: (0, 0)),
        ],
        out_specs=pl.BlockSpec((bm, embed), lambda i: (i, 0)),
        out_shape=jax.ShapeDtypeStruct((n_nodes, embed), jnp.float32),
    )(x, weight)

    pad = 64 * N_SUBCORES
    n_acc = ((n_nodes + pad - 1) // pad) * pad
    edge_flat = edge_index.reshape(2 * n_edges)
    partials = _make_sc_aggregate(n_acc, n_edges, embed)(h, edge_flat)

    out = pl.pallas_call(
        _add_body,
        grid=(n_nodes // bm,),
        in_specs=[pl.BlockSpec((N_CORES, bm, embed), lambda i: (0, i, 0))],
        out_specs=pl.BlockSpec((bm, embed), lambda i: (i, 0)),
        out_shape=jax.ShapeDtypeStruct((n_nodes, embed), jnp.float32),
    )(partials)
    return out
